# baseline (speedup 1.0000x reference)
import jax
import jax.numpy as jnp
from jax import lax
from jax.experimental import pallas as pl
from jax.experimental.pallas import tpu as pltpu
from jax.experimental.pallas import tpu_sc as plsc

VOCAB = 8192
NC, NS = 2, 16
NW = NC * NS
K = 4
ROWS_PER_W = 128
NCHUNK = ROWS_PER_W // K
NBUF = 3


def _body(idx_hbm, table_hbm, out_hbm, idx_v, buf_v, g0, g1, g2, o0, o1, o2):
    wid = lax.axis_index("s") * NC + lax.axis_index("c")
    row0 = wid * ROWS_PER_W

    pltpu.sync_copy(idx_hbm.at[wid], idx_v)

    gsem = (g0, g1, g2)
    osem = (o0, o1, o2)

    def g_start(c, b):
        pltpu.make_async_copy(
            table_hbm.at[idx_v.at[c]], buf_v.at[b], gsem[b]).start()

    def g_wait(b):
        pltpu.make_async_copy(
            table_hbm.at[idx_v.at[0]], buf_v.at[b], gsem[b]).wait()

    def o_start(c, b):
        pltpu.make_async_copy(
            buf_v.at[b], out_hbm.at[pl.ds(row0 + c * K, K)], osem[b]).start()

    def o_wait(b):
        pltpu.make_async_copy(
            buf_v.at[b], out_hbm.at[pl.ds(row0, K)], osem[b]).wait()

    def mid_chunk(c, b):
        o_wait((b + 1) % NBUF)
        g_start(c + 1, (b + 1) % NBUF)
        g_wait(b)
        o_start(c, b)

    g_start(0, 0)
    g_start(1, 1)
    g_wait(0)
    o_start(0, 0)
    g_start(2, 2)
    g_wait(1)
    o_start(1, 1)
    o_wait(0)
    g_start(3, 0)
    g_wait(2)
    o_start(2, 2)

    def loop_body(i, _):
        c = 3 * i + 3
        mid_chunk(c, 0)
        mid_chunk(c + 1, 1)
        mid_chunk(c + 2, 2)
        return _

    lax.fori_loop(0, (NCHUNK - 5) // 3, loop_body, None)

    mid_chunk(NCHUNK - 2, (NCHUNK - 2) % NBUF)
    b_last = (NCHUNK - 1) % NBUF
    o_wait((b_last + 1) % NBUF)
    g_wait(b_last)
    o_start(NCHUNK - 1, b_last)
    o_wait((NCHUNK - 2) % NBUF)
    o_wait(b_last)


_gather = pl.kernel(
    _body,
    out_type=jax.ShapeDtypeStruct((NW * ROWS_PER_W, VOCAB), jnp.float32),
    mesh=plsc.VectorSubcoreMesh(core_axis_name="c", subcore_axis_name="s"),
    scratch_types=[
        pltpu.VMEM((NCHUNK, K), jnp.int32),
        pltpu.VMEM((NBUF, K, VOCAB), jnp.float32),
        pltpu.SemaphoreType.DMA,
        pltpu.SemaphoreType.DMA,
        pltpu.SemaphoreType.DMA,
        pltpu.SemaphoreType.DMA,
        pltpu.SemaphoreType.DMA,
        pltpu.SemaphoreType.DMA,
    ],
)


def kernel(inputs, targets, table):
    del targets
    b, l = inputs.shape
    idx = inputs.astype(jnp.int32).reshape(NW, NCHUNK, K)
    out = _gather(idx, table)
    return out.reshape(b, l, VOCAB)


# speedup vs baseline: 1.0263x; 1.0263x over previous
import jax
import jax.numpy as jnp
from jax import lax
from jax.experimental import pallas as pl
from jax.experimental.pallas import tpu as pltpu
from jax.experimental.pallas import tpu_sc as plsc

VOCAB = 8192
NC, NS = 2, 16
NW = NC * NS
K = 2
ROWS_PER_W = 128
NCHUNK = ROWS_PER_W // K
NBUF = 3


def _body(idx_hbm, table_hbm, out_hbm, idx_v, buf_v, buf_s,
          g0, g1, g2, m0, m1, m2, o0, o1, o2):
    cid = lax.axis_index("c")
    sid = lax.axis_index("s")
    wid = sid * NC + cid
    row0 = wid * ROWS_PER_W

    pltpu.sync_copy(idx_hbm.at[wid], idx_v)

    gsem = (g0, g1, g2)
    msem = (m0, m1, m2)
    osem = (o0, o1, o2)

    def g_start(c, b):
        pltpu.make_async_copy(
            table_hbm.at[idx_v.at[c]], buf_v.at[b], gsem[b]).start()

    def g_wait(b):
        pltpu.make_async_copy(
            table_hbm.at[idx_v.at[0]], buf_v.at[b], gsem[b]).wait()

    def m_start(b):
        pltpu.make_async_copy(buf_v.at[b], buf_s.at[sid, b], msem[b]).start()

    def m_wait(b):
        pltpu.make_async_copy(buf_v.at[b], buf_s.at[sid, b], msem[b]).wait()

    def o_start(c, b):
        pltpu.make_async_copy(
            buf_s.at[sid, b], out_hbm.at[pl.ds(row0 + c * K, K)],
            osem[b]).start()

    def o_wait(b):
        pltpu.make_async_copy(
            buf_s.at[sid, b], out_hbm.at[pl.ds(row0, K)], osem[b]).wait()

    def chunk_head(g, b):
        g_start(g + 2, (b + 2) % NBUF)
        g_wait(b)
        m_start(b)
        m_wait(b)
        o_start(g, b)

    def chunk_mid(g, b):
        g_start(g + 2, (b + 2) % NBUF)
        g_wait(b)
        o_wait(b)
        m_start(b)
        m_wait(b)
        o_start(g, b)

    def chunk_tail(g, b):
        g_wait(b)
        o_wait(b)
        m_start(b)
        m_wait(b)
        o_start(g, b)

    g_start(0, 0)
    g_start(1, 1)
    chunk_head(0, 0)
    chunk_head(1, 1)
    chunk_head(2, 2)

    def loop_body(i, _):
        g = 3 * i + 3
        chunk_mid(g, 0)
        chunk_mid(g + 1, 1)
        chunk_mid(g + 2, 2)
        return _

    n_loop = (NCHUNK - 5) // 3
    lax.fori_loop(0, n_loop, loop_body, None)
    for g in range(3 * n_loop + 3, NCHUNK - 2):
        chunk_mid(g, g % NBUF)

    chunk_tail(NCHUNK - 2, (NCHUNK - 2) % NBUF)
    chunk_tail(NCHUNK - 1, (NCHUNK - 1) % NBUF)
    o_wait((NCHUNK - 3) % NBUF)
    o_wait((NCHUNK - 2) % NBUF)
    o_wait((NCHUNK - 1) % NBUF)


_gather = pl.kernel(
    _body,
    out_type=jax.ShapeDtypeStruct((NW * ROWS_PER_W, VOCAB), jnp.float32),
    mesh=plsc.VectorSubcoreMesh(core_axis_name="c", subcore_axis_name="s"),
    scratch_types=[
        pltpu.VMEM((NCHUNK, K), jnp.int32),
        pltpu.VMEM((NBUF, K, VOCAB), jnp.float32),
        pltpu.MemorySpace.VMEM_SHARED((NS, NBUF, K, VOCAB), jnp.float32),
        pltpu.SemaphoreType.DMA,
        pltpu.SemaphoreType.DMA,
        pltpu.SemaphoreType.DMA,
        pltpu.SemaphoreType.DMA,
        pltpu.SemaphoreType.DMA,
        pltpu.SemaphoreType.DMA,
        pltpu.SemaphoreType.DMA,
        pltpu.SemaphoreType.DMA,
        pltpu.SemaphoreType.DMA,
    ],
)


def kernel(inputs, targets, table):
    del targets
    b, l = inputs.shape
    idx = inputs.astype(jnp.int32).reshape(NW, NCHUNK, K)
    out = _gather(idx, table)
    return out.reshape(b, l, VOCAB)
